# baseline (speedup 1.0000x reference)
import functools

import jax
import jax.numpy as jnp
from jax import lax
from jax.experimental import pallas as pl
from jax.experimental.pallas import tpu as pltpu
from jax.experimental.pallas import tpu_sc as plsc

D_MODEL = 768
N_EXPERTS = 64
D_FF = 1024
N_TOKENS = 2048
LANES = 128

TM = 128
N_TILES = N_TOKENS // TM
P_PAIRS = N_TILES + N_EXPERTS

_NC = 2
_NS = 16
_NW = _NC * _NS
_BPW = N_TOKENS // _NW


def _router_body(x_ref, wr_ref, maskp_ref, logits_ref, sel_ref, wtop_ref):
    x = x_ref[...]
    logits = jnp.dot(x, wr_ref[...], preferred_element_type=jnp.float32)
    logits = logits * maskp_ref[...]
    logits_ref[...] = logits
    col = lax.broadcasted_iota(jnp.int32, (N_TOKENS, LANES), 1)
    scores = jnp.where(col > N_EXPERTS, -1e30, logits)
    m = jnp.max(scores, axis=1, keepdims=True)
    e = jnp.exp(scores - m)
    s = jnp.sum(e, axis=1, keepdims=True)
    wtop_ref[...] = 1.0 / s
    idx = jnp.where(scores >= m, col, LANES)
    sel_ref[...] = jnp.min(idx, axis=1, keepdims=True)


def _router(x, W_router, mask):
    wr_pad = jnp.zeros((D_MODEL, LANES), jnp.float32).at[:, :N_EXPERTS].set(W_router)
    maskp = jnp.concatenate(
        [mask.astype(jnp.float32), jnp.ones((LANES - N_EXPERTS,), jnp.float32)]
    ).reshape(1, LANES)
    return pl.pallas_call(
        _router_body,
        out_shape=(
            jax.ShapeDtypeStruct((N_TOKENS, LANES), jnp.float32),
            jax.ShapeDtypeStruct((N_TOKENS, 1), jnp.int32),
            jax.ShapeDtypeStruct((N_TOKENS, 1), jnp.float32),
        ),
    )(x, wr_pad, maskp)


def _route_metadata(sel):
    onehot = (sel[:, None] == jnp.arange(N_EXPERTS + 1, dtype=jnp.int32)[None, :])
    cum = jnp.cumsum(onehot.astype(jnp.int32), axis=0)
    g = cum[-1]
    ends = jnp.cumsum(g)
    starts = ends - g
    rank = jnp.take_along_axis(cum, sel[:, None], axis=1)[:, 0] - 1
    pos = starts[sel] + rank
    nonempty = g > 0
    t_lo = starts // TM
    t_hi = (ends - 1) // TM
    c = jnp.where(nonempty, t_hi - t_lo + 1, 0)
    ccum_end = jnp.cumsum(c)
    ccum_start = ccum_end - c
    total = ccum_end[-1]
    p = jnp.arange(P_PAIRS, dtype=jnp.int32)
    e = jnp.clip(jnp.searchsorted(ccum_end, p, side="right"), 0, N_EXPERTS).astype(jnp.int32)
    j = p - ccum_start[e]
    t = jnp.clip(t_lo[e] + j, 0, N_TILES - 1)
    is_pad = p >= total
    tid = jnp.where(is_pad, N_TILES - 1, t)
    real = (~is_pad) & (e < N_EXPERTS)
    rs = jnp.where(real, jnp.clip(starts[e] - tid * TM, 0, TM), 0)
    re = jnp.where(real, jnp.clip(ends[e] - tid * TM, 0, TM), 0)
    el = jnp.minimum(e, N_EXPERTS - 1)
    flag = (real & (j == 0)).astype(jnp.int32)
    slot = jnp.maximum(jnp.cumsum(flag) - 1, 0).astype(jnp.int32) % NBUF
    return pos, tid, el, rs, re, flag, slot


NBUF = 4


def _gmm_body(tile_ref, exp_ref, rs_ref, re_ref, flag_ref, slot_ref,
              xs_ref, b1_ref, b2_ref, w1_hbm, w2_hbm, out_ref,
              w1_buf, w2_buf, sem1, sem2):
    p = pl.program_id(0)

    def fetch(step):
        s = slot_ref[step]
        e = exp_ref[step]
        pltpu.make_async_copy(w1_hbm.at[e], w1_buf.at[s], sem1.at[s]).start()
        pltpu.make_async_copy(w2_hbm.at[e], w2_buf.at[s], sem2.at[s]).start()

    @pl.when(p == 0)
    def _prologue():
        for i in range(NBUF - 1):
            @pl.when(flag_ref[i] == 1)
            def _f(i=i):
                fetch(i)

    q = jnp.minimum(p + NBUF - 1, P_PAIRS - 1)

    @pl.when((p + NBUF - 1 < P_PAIRS) & (flag_ref[q] == 1))
    def _issue():
        fetch(q)

    s = slot_ref[p]
    e = exp_ref[p]

    @pl.when(flag_ref[p] == 1)
    def _wait():
        pltpu.make_async_copy(w1_hbm.at[e], w1_buf.at[s], sem1.at[s]).wait()
        pltpu.make_async_copy(w2_hbm.at[e], w2_buf.at[s], sem2.at[s]).wait()

    first = jnp.logical_or(p == 0, tile_ref[p] != tile_ref[jnp.maximum(p - 1, 0)])
    rs = rs_ref[p]
    re = re_ref[p]

    @pl.when(first)
    def _init():
        out_ref[...] = jnp.zeros_like(out_ref)

    @pl.when(re > rs)
    def _compute():
        rows = lax.broadcasted_iota(jnp.int32, (TM, 1), 0)
        scale = jnp.where((rows >= rs) & (rows < re), 1.0, 0.0)
        h = jnp.dot(xs_ref[...], w1_buf[s], preferred_element_type=jnp.float32)
        h = jax.nn.gelu(h + b1_ref[pl.ds(e, 1), :])
        o = jnp.dot(h, w2_buf[s], preferred_element_type=jnp.float32)
        o = o + b2_ref[pl.ds(e, 1), :]
        out_ref[...] += scale * o


def _gmm(tid, el, rs, re, flag, slot, xs, W1, b1, W2, b2):
    grid_spec = pltpu.PrefetchScalarGridSpec(
        num_scalar_prefetch=6,
        grid=(P_PAIRS,),
        in_specs=[
            pl.BlockSpec((TM, D_MODEL), lambda p, t, e, a, b, f, s: (t[p], 0)),
            pl.BlockSpec((N_EXPERTS, D_FF), lambda p, t, e, a, b, f, s: (0, 0)),
            pl.BlockSpec((N_EXPERTS, D_MODEL), lambda p, t, e, a, b, f, s: (0, 0)),
            pl.BlockSpec(memory_space=pl.ANY),
            pl.BlockSpec(memory_space=pl.ANY),
        ],
        out_specs=pl.BlockSpec((TM, D_MODEL), lambda p, t, e, a, b, f, s: (t[p], 0)),
        scratch_shapes=[
            pltpu.VMEM((NBUF, D_MODEL, D_FF), jnp.float32),
            pltpu.VMEM((NBUF, D_FF, D_MODEL), jnp.float32),
            pltpu.SemaphoreType.DMA((NBUF,)),
            pltpu.SemaphoreType.DMA((NBUF,)),
        ],
    )
    return pl.pallas_call(
        _gmm_body,
        grid_spec=grid_spec,
        out_shape=jax.ShapeDtypeStruct((N_TOKENS, D_MODEL), jnp.float32),
        compiler_params=pltpu.CompilerParams(
            dimension_semantics=("arbitrary",),
        ),
    )(tid, el, rs, re, flag, slot, xs, b1, b2, W1, W2)


def _sc_gather_body(table_hbm, idx_hbm, out_hbm, idx_v, rows_v, sem):
    wid = lax.axis_index("s") * _NC + lax.axis_index("c")
    base = wid * _BPW
    pltpu.sync_copy(idx_hbm.at[pl.ds(base, _BPW)], idx_v)
    pltpu.async_copy(table_hbm.at[idx_v], rows_v, sem).wait()
    pltpu.sync_copy(rows_v, out_hbm.at[pl.ds(base, _BPW)])


def _sc_scatter_body(table_hbm, idx_hbm, out_hbm, idx_v, rows_v, sem):
    wid = lax.axis_index("s") * _NC + lax.axis_index("c")
    base = wid * _BPW
    pltpu.sync_copy(idx_hbm.at[pl.ds(base, _BPW)], idx_v)
    pltpu.sync_copy(table_hbm.at[pl.ds(base, _BPW)], rows_v)
    pltpu.async_copy(rows_v, out_hbm.at[idx_v], sem).wait()


def _sc_scatter(table, idx):
    mesh = plsc.VectorSubcoreMesh(
        core_axis_name="c", subcore_axis_name="s", num_cores=_NC, num_subcores=_NS)
    k = functools.partial(
        pl.kernel,
        mesh=mesh,
        out_type=jax.ShapeDtypeStruct((N_TOKENS, D_MODEL), jnp.float32),
        scratch_types=[
            pltpu.VMEM((_BPW,), jnp.int32),
            pltpu.VMEM((_BPW, D_MODEL), jnp.float32),
            pltpu.SemaphoreType.DMA,
        ],
    )(_sc_scatter_body)
    return k(table, idx)


def _sc_gather(table, idx):
    mesh = plsc.VectorSubcoreMesh(
        core_axis_name="c", subcore_axis_name="s", num_cores=_NC, num_subcores=_NS)
    k = functools.partial(
        pl.kernel,
        mesh=mesh,
        out_type=jax.ShapeDtypeStruct((N_TOKENS, D_MODEL), jnp.float32),
        scratch_types=[
            pltpu.VMEM((_BPW,), jnp.int32),
            pltpu.VMEM((_BPW, D_MODEL), jnp.float32),
            pltpu.SemaphoreType.DMA,
        ],
    )(_sc_gather_body)
    return k(table, idx)


def kernel(inputs, mask, W_router, W1, b1, W2, b2):
    x = inputs.reshape(N_TOKENS, D_MODEL)
    logits_pad, sel2d, wtop = _router(x, W_router, mask)
    sel = sel2d[:, 0]
    pos, tid, el, rs, re, flag, slot = _route_metadata(sel)
    xs = _sc_scatter(x, pos)
    ys = _gmm(tid, el, rs, re, flag, slot, xs, W1, b1, W2, b2)
    out = _sc_gather(ys, pos)
    results = (out * wtop).reshape(inputs.shape)
    return (results, logits_pad[:, :N_EXPERTS + 1], sel2d)


# speedup vs baseline: 1.2406x; 1.2406x over previous
import functools

import jax
import jax.numpy as jnp
from jax import lax
from jax.experimental import pallas as pl
from jax.experimental.pallas import tpu as pltpu
from jax.experimental.pallas import tpu_sc as plsc

D_MODEL = 768
N_EXPERTS = 64
D_FF = 1024
N_TOKENS = 2048
LANES = 128

TM = 128
N_TILES = N_TOKENS // TM
P_PAIRS = N_TILES + N_EXPERTS

_NC = 2
_NS = 16
_NW = _NC * _NS
_BPW = N_TOKENS // _NW


CHUNK = 256


def _router_body(x_ref, wr_ref, maskp_ref, tril_ref, mstrict_ref, trilp_ref,
                 logits_ref, sel_ref, wtop_ref, pos_ref, meta_ref):
    x = x_ref[...]
    logits = jnp.dot(x, wr_ref[...], preferred_element_type=jnp.float32)
    logits = logits * maskp_ref[...]
    logits_ref[...] = logits
    col = lax.broadcasted_iota(jnp.int32, (N_TOKENS, LANES), 1)
    scores = jnp.where(col > N_EXPERTS, -1e30, logits)
    m = jnp.max(scores, axis=1, keepdims=True)
    ex = jnp.exp(scores - m)
    s = jnp.sum(ex, axis=1, keepdims=True)
    wtop_ref[...] = 1.0 / s
    idx = jnp.where(scores >= m, col, LANES)
    sel = jnp.min(idx, axis=1, keepdims=True)
    sel_ref[...] = sel

    onehot = (col == sel).astype(jnp.float32)
    tril = tril_ref[...]
    carry = jnp.zeros((1, LANES), jnp.float32)
    ranks = []
    for cidx in range(N_TOKENS // CHUNK):
        blk = onehot[cidx * CHUNK:(cidx + 1) * CHUNK]
        cumc = jnp.dot(tril, blk, preferred_element_type=jnp.float32) + carry
        ranks.append(jnp.sum(cumc * blk, axis=1, keepdims=True))
        carry = cumc[CHUNK - 1:CHUNK, :]
    g = carry
    rank = jnp.concatenate(ranks, axis=0) - 1.0
    starts = jnp.dot(g, mstrict_ref[...], preferred_element_type=jnp.float32,
                     precision=lax.Precision.HIGHEST)
    pos = jnp.sum(starts * onehot, axis=1, keepdims=True) + rank
    pos_ref[...] = pos.astype(jnp.int32)

    ends = starts + g
    lane = col[0:1, :].astype(jnp.float32)
    t_lo = jnp.floor(starts * (1.0 / TM))
    t_hi = jnp.floor((ends - 1.0) * (1.0 / TM))
    cnt = jnp.where(g > 0, t_hi - t_lo + 1.0, 0.0)
    ccum_end = jnp.dot(cnt, mstrict_ref[...], preferred_element_type=jnp.float32,
                       precision=lax.Precision.HIGHEST) + cnt
    ccum_start = ccum_end - cnt
    total = jnp.sum(cnt, axis=1, keepdims=True)
    prow = lax.broadcasted_iota(jnp.int32, (P_PAIRS, 1), 0).astype(jnp.float32)
    lane_ok = lane <= N_EXPERTS
    e_of_p = jnp.sum(((ccum_end <= prow) & lane_ok).astype(jnp.float32),
                     axis=1, keepdims=True)
    e = jnp.minimum(e_of_p, float(N_EXPERTS))
    onehot_e = (lane == e).astype(jnp.float32)

    def gath(row):
        return jnp.sum(onehot_e * row, axis=1, keepdims=True)

    j = prow - gath(ccum_start)
    t = jnp.clip(gath(t_lo) + j, 0.0, float(N_TILES - 1))
    is_pad = prow >= total
    tid = jnp.where(is_pad, float(N_TILES - 1), t)
    real = jnp.logical_and(~is_pad, e < float(N_EXPERTS))
    rs = jnp.where(real, jnp.clip(gath(starts) - tid * TM, 0.0, float(TM)), 0.0)
    re = jnp.where(real, jnp.clip(gath(ends) - tid * TM, 0.0, float(TM)), 0.0)
    el = jnp.minimum(e, float(N_EXPERTS - 1))
    flag = jnp.where(jnp.logical_and(real, j == 0.0), 1.0, 0.0)
    cumflag = jnp.dot(trilp_ref[...], flag, preferred_element_type=jnp.float32,
                      precision=lax.Precision.HIGHEST)
    slotf = jnp.maximum(cumflag - 1.0, 0.0)
    slot = slotf - NBUF * jnp.floor(slotf * (1.0 / NBUF))
    mcol = lax.broadcasted_iota(jnp.int32, (P_PAIRS, 8), 1)
    fields = [tid, el, rs, re, flag, slot]
    meta = jnp.zeros((P_PAIRS, 8), jnp.float32)
    for k, f in enumerate(fields):
        meta = meta + jnp.where(mcol == k, f, 0.0)
    meta_ref[...] = meta.astype(jnp.int32)


def _router(x, W_router, mask):
    wr_pad = jnp.zeros((D_MODEL, LANES), jnp.float32).at[:, :N_EXPERTS].set(W_router)
    maskp = jnp.concatenate(
        [mask.astype(jnp.float32), jnp.ones((LANES - N_EXPERTS,), jnp.float32)]
    ).reshape(1, LANES)
    r = jnp.arange(CHUNK)
    tril = (r[:, None] >= r[None, :]).astype(jnp.float32)
    l = jnp.arange(LANES)
    mstrict = (l[:, None] < l[None, :]).astype(jnp.float32)
    q = jnp.arange(P_PAIRS)
    trilp = (q[:, None] >= q[None, :]).astype(jnp.float32)
    return pl.pallas_call(
        _router_body,
        out_shape=(
            jax.ShapeDtypeStruct((N_TOKENS, LANES), jnp.float32),
            jax.ShapeDtypeStruct((N_TOKENS, 1), jnp.int32),
            jax.ShapeDtypeStruct((N_TOKENS, 1), jnp.float32),
            jax.ShapeDtypeStruct((N_TOKENS, 1), jnp.int32),
            jax.ShapeDtypeStruct((P_PAIRS, 8), jnp.int32),
        ),
    )(x, wr_pad, maskp, tril, mstrict, trilp)


def _route_metadata(sel):
    onehot = (sel[:, None] == jnp.arange(N_EXPERTS + 1, dtype=jnp.int32)[None, :])
    cum = jnp.cumsum(onehot.astype(jnp.int32), axis=0)
    g = cum[-1]
    ends = jnp.cumsum(g)
    starts = ends - g
    rank = jnp.take_along_axis(cum, sel[:, None], axis=1)[:, 0] - 1
    pos = starts[sel] + rank
    nonempty = g > 0
    t_lo = starts // TM
    t_hi = (ends - 1) // TM
    c = jnp.where(nonempty, t_hi - t_lo + 1, 0)
    ccum_end = jnp.cumsum(c)
    ccum_start = ccum_end - c
    total = ccum_end[-1]
    p = jnp.arange(P_PAIRS, dtype=jnp.int32)
    e = jnp.clip(jnp.searchsorted(ccum_end, p, side="right"), 0, N_EXPERTS).astype(jnp.int32)
    j = p - ccum_start[e]
    t = jnp.clip(t_lo[e] + j, 0, N_TILES - 1)
    is_pad = p >= total
    tid = jnp.where(is_pad, N_TILES - 1, t)
    real = (~is_pad) & (e < N_EXPERTS)
    rs = jnp.where(real, jnp.clip(starts[e] - tid * TM, 0, TM), 0)
    re = jnp.where(real, jnp.clip(ends[e] - tid * TM, 0, TM), 0)
    el = jnp.minimum(e, N_EXPERTS - 1)
    flag = (real & (j == 0)).astype(jnp.int32)
    slot = jnp.maximum(jnp.cumsum(flag) - 1, 0).astype(jnp.int32) % NBUF
    return pos, tid, el, rs, re, flag, slot


NBUF = 4


def _gmm_body(meta_ref,
              xs_ref, b1_ref, b2_ref, w1_hbm, w2_hbm, out_ref,
              w1_buf, w2_buf, sem1, sem2):
    p = pl.program_id(0)

    def fetch(step):
        s = meta_ref[step, 5]
        e = meta_ref[step, 1]
        pltpu.make_async_copy(w1_hbm.at[e], w1_buf.at[s], sem1.at[s]).start()
        pltpu.make_async_copy(w2_hbm.at[e], w2_buf.at[s], sem2.at[s]).start()

    @pl.when(p == 0)
    def _prologue():
        for i in range(NBUF - 1):
            @pl.when(meta_ref[i, 4] == 1)
            def _f(i=i):
                fetch(i)

    q = jnp.minimum(p + NBUF - 1, P_PAIRS - 1)

    @pl.when((p + NBUF - 1 < P_PAIRS) & (meta_ref[q, 4] == 1))
    def _issue():
        fetch(q)

    s = meta_ref[p, 5]
    e = meta_ref[p, 1]

    @pl.when(meta_ref[p, 4] == 1)
    def _wait():
        pltpu.make_async_copy(w1_hbm.at[e], w1_buf.at[s], sem1.at[s]).wait()
        pltpu.make_async_copy(w2_hbm.at[e], w2_buf.at[s], sem2.at[s]).wait()

    first = jnp.logical_or(p == 0, meta_ref[p, 0] != meta_ref[jnp.maximum(p - 1, 0), 0])
    rs = meta_ref[p, 2]
    re = meta_ref[p, 3]

    @pl.when(first)
    def _init():
        out_ref[...] = jnp.zeros_like(out_ref)

    @pl.when(re > rs)
    def _compute():
        rows = lax.broadcasted_iota(jnp.int32, (TM, 1), 0)
        scale = jnp.where((rows >= rs) & (rows < re), 1.0, 0.0)
        h = jnp.dot(xs_ref[...], w1_buf[s], preferred_element_type=jnp.float32)
        h = jax.nn.gelu(h + b1_ref[pl.ds(e, 1), :])
        o = jnp.dot(h, w2_buf[s], preferred_element_type=jnp.float32)
        o = o + b2_ref[pl.ds(e, 1), :]
        out_ref[...] += scale * o


def _gmm(meta, xs, W1, b1, W2, b2):
    grid_spec = pltpu.PrefetchScalarGridSpec(
        num_scalar_prefetch=1,
        grid=(P_PAIRS,),
        in_specs=[
            pl.BlockSpec((TM, D_MODEL), lambda p, m: (m[p, 0], 0)),
            pl.BlockSpec((N_EXPERTS, D_FF), lambda p, m: (0, 0)),
            pl.BlockSpec((N_EXPERTS, D_MODEL), lambda p, m: (0, 0)),
            pl.BlockSpec(memory_space=pl.ANY),
            pl.BlockSpec(memory_space=pl.ANY),
        ],
        out_specs=pl.BlockSpec((TM, D_MODEL), lambda p, m: (m[p, 0], 0)),
        scratch_shapes=[
            pltpu.VMEM((NBUF, D_MODEL, D_FF), jnp.float32),
            pltpu.VMEM((NBUF, D_FF, D_MODEL), jnp.float32),
            pltpu.SemaphoreType.DMA((NBUF,)),
            pltpu.SemaphoreType.DMA((NBUF,)),
        ],
    )
    return pl.pallas_call(
        _gmm_body,
        grid_spec=grid_spec,
        out_shape=jax.ShapeDtypeStruct((N_TOKENS, D_MODEL), jnp.float32),
        compiler_params=pltpu.CompilerParams(
            dimension_semantics=("arbitrary",),
        ),
    )(meta, xs, b1, b2, W1, W2)


def _sc_gather_body(table_hbm, idx_hbm, out_hbm, idx_v, rows_v, sem):
    wid = lax.axis_index("s") * _NC + lax.axis_index("c")
    base = wid * _BPW
    pltpu.sync_copy(idx_hbm.at[pl.ds(base, _BPW)], idx_v)
    pltpu.async_copy(table_hbm.at[idx_v], rows_v, sem).wait()
    pltpu.sync_copy(rows_v, out_hbm.at[pl.ds(base, _BPW)])


def _sc_scatter_body(table_hbm, idx_hbm, out_hbm, idx_v, rows_v, sem):
    wid = lax.axis_index("s") * _NC + lax.axis_index("c")
    base = wid * _BPW
    pltpu.sync_copy(idx_hbm.at[pl.ds(base, _BPW)], idx_v)
    pltpu.sync_copy(table_hbm.at[pl.ds(base, _BPW)], rows_v)
    pltpu.async_copy(rows_v, out_hbm.at[idx_v], sem).wait()


def _sc_scatter(table, idx):
    mesh = plsc.VectorSubcoreMesh(
        core_axis_name="c", subcore_axis_name="s", num_cores=_NC, num_subcores=_NS)
    k = functools.partial(
        pl.kernel,
        mesh=mesh,
        out_type=jax.ShapeDtypeStruct((N_TOKENS, D_MODEL), jnp.float32),
        scratch_types=[
            pltpu.VMEM((_BPW,), jnp.int32),
            pltpu.VMEM((_BPW, D_MODEL), jnp.float32),
            pltpu.SemaphoreType.DMA,
        ],
    )(_sc_scatter_body)
    return k(table, idx)


def _sc_gather(table, idx):
    mesh = plsc.VectorSubcoreMesh(
        core_axis_name="c", subcore_axis_name="s", num_cores=_NC, num_subcores=_NS)
    k = functools.partial(
        pl.kernel,
        mesh=mesh,
        out_type=jax.ShapeDtypeStruct((N_TOKENS, D_MODEL), jnp.float32),
        scratch_types=[
            pltpu.VMEM((_BPW,), jnp.int32),
            pltpu.VMEM((_BPW, D_MODEL), jnp.float32),
            pltpu.SemaphoreType.DMA,
        ],
    )(_sc_gather_body)
    return k(table, idx)


def kernel(inputs, mask, W_router, W1, b1, W2, b2):
    x = inputs.reshape(N_TOKENS, D_MODEL)
    logits_pad, sel2d, wtop, pos2d, meta = _router(x, W_router, mask)
    pos = pos2d.reshape(N_TOKENS)
    xs = _sc_scatter(x, pos)
    ys = _gmm(meta, xs, W1, b1, W2, b2)
    out = _sc_gather(ys, pos)
    results = (out * wtop).reshape(inputs.shape)
    return (results, logits_pad[:, :N_EXPERTS + 1], sel2d)


# speedup vs baseline: 1.2550x; 1.0116x over previous
import functools

import numpy as np

import jax
import jax.numpy as jnp
from jax import lax
from jax.experimental import pallas as pl
from jax.experimental.pallas import tpu as pltpu
from jax.experimental.pallas import tpu_sc as plsc

D_MODEL = 768
N_EXPERTS = 64
D_FF = 1024
N_TOKENS = 2048
LANES = 128

TM = 128
N_TILES = N_TOKENS // TM
P_PAIRS = N_TILES + N_EXPERTS

_NC = 2
_NS = 16
_NW = _NC * _NS
_BPW = N_TOKENS // _NW


CHUNK = 256


def _router_body(x_ref, wr_ref, maskp_ref, tril_ref, mstrict_ref, trilp_ref,
                 logits_ref, sel_ref, wtop_ref, pos_ref, meta_ref):
    x = x_ref[...]
    logits = jnp.dot(x, wr_ref[...], preferred_element_type=jnp.float32)
    logits = logits * maskp_ref[...]
    logits_ref[...] = logits
    col = lax.broadcasted_iota(jnp.int32, (N_TOKENS, LANES), 1)
    scores = jnp.where(col > N_EXPERTS, -1e30, logits)
    m = jnp.max(scores, axis=1, keepdims=True)
    ex = jnp.exp(scores - m)
    s = jnp.sum(ex, axis=1, keepdims=True)
    wtop_ref[...] = 1.0 / s
    idx = jnp.where(scores >= m, col, LANES)
    sel = jnp.min(idx, axis=1, keepdims=True)
    sel_ref[...] = sel

    onehot = (col == sel).astype(jnp.float32)
    tril = tril_ref[...]
    carry = jnp.zeros((1, LANES), jnp.float32)
    ranks = []
    for cidx in range(N_TOKENS // CHUNK):
        blk = onehot[cidx * CHUNK:(cidx + 1) * CHUNK]
        cumc = jnp.dot(tril, blk, preferred_element_type=jnp.float32) + carry
        ranks.append(jnp.sum(cumc * blk, axis=1, keepdims=True))
        carry = cumc[CHUNK - 1:CHUNK, :]
    g = carry
    rank = jnp.concatenate(ranks, axis=0) - 1.0
    starts = jnp.dot(g, mstrict_ref[...], preferred_element_type=jnp.float32,
                     precision=lax.Precision.HIGHEST)
    pos = jnp.sum(starts * onehot, axis=1, keepdims=True) + rank
    pos_ref[...] = pos.astype(jnp.int32)

    ends = starts + g
    lane = col[0:1, :].astype(jnp.float32)
    t_lo = jnp.floor(starts * (1.0 / TM))
    t_hi = jnp.floor((ends - 1.0) * (1.0 / TM))
    cnt = jnp.where(g > 0, t_hi - t_lo + 1.0, 0.0)
    ccum_end = jnp.dot(cnt, mstrict_ref[...], preferred_element_type=jnp.float32,
                       precision=lax.Precision.HIGHEST) + cnt
    ccum_start = ccum_end - cnt
    total = jnp.sum(cnt, axis=1, keepdims=True)
    prow = lax.broadcasted_iota(jnp.int32, (P_PAIRS, 1), 0).astype(jnp.float32)
    lane_ok = lane <= N_EXPERTS
    e_of_p = jnp.sum(((ccum_end <= prow) & lane_ok).astype(jnp.float32),
                     axis=1, keepdims=True)
    e = jnp.minimum(e_of_p, float(N_EXPERTS))
    onehot_e = (lane == e).astype(jnp.float32)

    def gath(row):
        return jnp.sum(onehot_e * row, axis=1, keepdims=True)

    j = prow - gath(ccum_start)
    t = jnp.clip(gath(t_lo) + j, 0.0, float(N_TILES - 1))
    is_pad = prow >= total
    tid = jnp.where(is_pad, float(N_TILES - 1), t)
    real = jnp.logical_and(~is_pad, e < float(N_EXPERTS))
    rs = jnp.where(real, jnp.clip(gath(starts) - tid * TM, 0.0, float(TM)), 0.0)
    re = jnp.where(real, jnp.clip(gath(ends) - tid * TM, 0.0, float(TM)), 0.0)
    el = jnp.minimum(e, float(N_EXPERTS - 1))
    flag = jnp.where(jnp.logical_and(real, j == 0.0), 1.0, 0.0)
    cumflag = jnp.dot(trilp_ref[...], flag, preferred_element_type=jnp.float32,
                      precision=lax.Precision.HIGHEST)
    slotf = jnp.maximum(cumflag - 1.0, 0.0)
    slot = slotf - NBUF * jnp.floor(slotf * (1.0 / NBUF))
    mcol = lax.broadcasted_iota(jnp.int32, (P_PAIRS, 8), 1)
    fields = [tid, el, rs, re, flag, slot]
    meta = jnp.zeros((P_PAIRS, 8), jnp.float32)
    for k, f in enumerate(fields):
        meta = meta + jnp.where(mcol == k, f, 0.0)
    meta_ref[...] = meta.astype(jnp.int32)


def _router(x, W_router, mask):
    wr_pad = jnp.zeros((D_MODEL, LANES), jnp.float32).at[:, :N_EXPERTS].set(W_router)
    maskp = jnp.concatenate(
        [mask.astype(jnp.float32), jnp.ones((LANES - N_EXPERTS,), jnp.float32)]
    ).reshape(1, LANES)
    tril = jnp.asarray(np.tril(np.ones((CHUNK, CHUNK), np.float32)))
    mstrict = jnp.asarray(
        np.triu(np.ones((LANES, LANES), np.float32), 1))
    trilp = jnp.asarray(np.tril(np.ones((P_PAIRS, P_PAIRS), np.float32)))
    return pl.pallas_call(
        _router_body,
        out_shape=(
            jax.ShapeDtypeStruct((N_TOKENS, LANES), jnp.float32),
            jax.ShapeDtypeStruct((N_TOKENS, 1), jnp.int32),
            jax.ShapeDtypeStruct((N_TOKENS, 1), jnp.float32),
            jax.ShapeDtypeStruct((N_TOKENS, 1), jnp.int32),
            jax.ShapeDtypeStruct((P_PAIRS, 8), jnp.int32),
        ),
    )(x, wr_pad, maskp, tril, mstrict, trilp)


def _route_metadata(sel):
    onehot = (sel[:, None] == jnp.arange(N_EXPERTS + 1, dtype=jnp.int32)[None, :])
    cum = jnp.cumsum(onehot.astype(jnp.int32), axis=0)
    g = cum[-1]
    ends = jnp.cumsum(g)
    starts = ends - g
    rank = jnp.take_along_axis(cum, sel[:, None], axis=1)[:, 0] - 1
    pos = starts[sel] + rank
    nonempty = g > 0
    t_lo = starts // TM
    t_hi = (ends - 1) // TM
    c = jnp.where(nonempty, t_hi - t_lo + 1, 0)
    ccum_end = jnp.cumsum(c)
    ccum_start = ccum_end - c
    total = ccum_end[-1]
    p = jnp.arange(P_PAIRS, dtype=jnp.int32)
    e = jnp.clip(jnp.searchsorted(ccum_end, p, side="right"), 0, N_EXPERTS).astype(jnp.int32)
    j = p - ccum_start[e]
    t = jnp.clip(t_lo[e] + j, 0, N_TILES - 1)
    is_pad = p >= total
    tid = jnp.where(is_pad, N_TILES - 1, t)
    real = (~is_pad) & (e < N_EXPERTS)
    rs = jnp.where(real, jnp.clip(starts[e] - tid * TM, 0, TM), 0)
    re = jnp.where(real, jnp.clip(ends[e] - tid * TM, 0, TM), 0)
    el = jnp.minimum(e, N_EXPERTS - 1)
    flag = (real & (j == 0)).astype(jnp.int32)
    slot = jnp.maximum(jnp.cumsum(flag) - 1, 0).astype(jnp.int32) % NBUF
    return pos, tid, el, rs, re, flag, slot


NBUF = 4


def _gmm_body(meta_ref,
              xs_ref, b1_ref, b2_ref, w1_hbm, w2_hbm, out_ref,
              w1_buf, w2_buf, sem1, sem2):
    p = pl.program_id(0)

    def fetch(step):
        s = meta_ref[step, 5]
        e = meta_ref[step, 1]
        pltpu.make_async_copy(w1_hbm.at[e], w1_buf.at[s], sem1.at[s]).start()
        pltpu.make_async_copy(w2_hbm.at[e], w2_buf.at[s], sem2.at[s]).start()

    @pl.when(p == 0)
    def _prologue():
        for i in range(NBUF - 1):
            @pl.when(meta_ref[i, 4] == 1)
            def _f(i=i):
                fetch(i)

    q = jnp.minimum(p + NBUF - 1, P_PAIRS - 1)

    @pl.when((p + NBUF - 1 < P_PAIRS) & (meta_ref[q, 4] == 1))
    def _issue():
        fetch(q)

    s = meta_ref[p, 5]
    e = meta_ref[p, 1]

    @pl.when(meta_ref[p, 4] == 1)
    def _wait():
        pltpu.make_async_copy(w1_hbm.at[e], w1_buf.at[s], sem1.at[s]).wait()
        pltpu.make_async_copy(w2_hbm.at[e], w2_buf.at[s], sem2.at[s]).wait()

    first = jnp.logical_or(p == 0, meta_ref[p, 0] != meta_ref[jnp.maximum(p - 1, 0), 0])
    rs = meta_ref[p, 2]
    re = meta_ref[p, 3]

    @pl.when(first)
    def _init():
        out_ref[...] = jnp.zeros_like(out_ref)

    @pl.when(re > rs)
    def _compute():
        rows = lax.broadcasted_iota(jnp.int32, (TM, 1), 0)
        scale = jnp.where((rows >= rs) & (rows < re), 1.0, 0.0)
        h = jnp.dot(xs_ref[...], w1_buf[s], preferred_element_type=jnp.float32)
        h = jax.nn.gelu(h + b1_ref[pl.ds(e, 1), :])
        o = jnp.dot(h, w2_buf[s], preferred_element_type=jnp.float32)
        o = o + b2_ref[pl.ds(e, 1), :]
        out_ref[...] += scale * o


def _gmm(meta, xs, W1, b1, W2, b2):
    grid_spec = pltpu.PrefetchScalarGridSpec(
        num_scalar_prefetch=1,
        grid=(P_PAIRS,),
        in_specs=[
            pl.BlockSpec((TM, D_MODEL), lambda p, m: (m[p, 0], 0)),
            pl.BlockSpec((N_EXPERTS, D_FF), lambda p, m: (0, 0)),
            pl.BlockSpec((N_EXPERTS, D_MODEL), lambda p, m: (0, 0)),
            pl.BlockSpec(memory_space=pl.ANY),
            pl.BlockSpec(memory_space=pl.ANY),
        ],
        out_specs=pl.BlockSpec((TM, D_MODEL), lambda p, m: (m[p, 0], 0)),
        scratch_shapes=[
            pltpu.VMEM((NBUF, D_MODEL, D_FF), jnp.float32),
            pltpu.VMEM((NBUF, D_FF, D_MODEL), jnp.float32),
            pltpu.SemaphoreType.DMA((NBUF,)),
            pltpu.SemaphoreType.DMA((NBUF,)),
        ],
    )
    return pl.pallas_call(
        _gmm_body,
        grid_spec=grid_spec,
        out_shape=jax.ShapeDtypeStruct((N_TOKENS, D_MODEL), jnp.float32),
        compiler_params=pltpu.CompilerParams(
            dimension_semantics=("arbitrary",),
        ),
    )(meta, xs, b1, b2, W1, W2)


def _sc_gather_body(table_hbm, idx_hbm, out_hbm, idx_v, rows_v, sem):
    wid = lax.axis_index("s") * _NC + lax.axis_index("c")
    base = wid * _BPW
    pltpu.sync_copy(idx_hbm.at[pl.ds(base, _BPW)], idx_v)
    pltpu.async_copy(table_hbm.at[idx_v], rows_v, sem).wait()
    pltpu.sync_copy(rows_v, out_hbm.at[pl.ds(base, _BPW)])


def _sc_scatter_body(table_hbm, idx_hbm, out_hbm, idx_v, rows_v, sem):
    wid = lax.axis_index("s") * _NC + lax.axis_index("c")
    base = wid * _BPW
    pltpu.sync_copy(idx_hbm.at[pl.ds(base, _BPW)], idx_v)
    pltpu.sync_copy(table_hbm.at[pl.ds(base, _BPW)], rows_v)
    pltpu.async_copy(rows_v, out_hbm.at[idx_v], sem).wait()


def _sc_scatter(table, idx):
    mesh = plsc.VectorSubcoreMesh(
        core_axis_name="c", subcore_axis_name="s", num_cores=_NC, num_subcores=_NS)
    k = functools.partial(
        pl.kernel,
        mesh=mesh,
        out_type=jax.ShapeDtypeStruct((N_TOKENS, D_MODEL), jnp.float32),
        scratch_types=[
            pltpu.VMEM((_BPW,), jnp.int32),
            pltpu.VMEM((_BPW, D_MODEL), jnp.float32),
            pltpu.SemaphoreType.DMA,
        ],
    )(_sc_scatter_body)
    return k(table, idx)


def _sc_gather(table, idx):
    mesh = plsc.VectorSubcoreMesh(
        core_axis_name="c", subcore_axis_name="s", num_cores=_NC, num_subcores=_NS)
    k = functools.partial(
        pl.kernel,
        mesh=mesh,
        out_type=jax.ShapeDtypeStruct((N_TOKENS, D_MODEL), jnp.float32),
        scratch_types=[
            pltpu.VMEM((_BPW,), jnp.int32),
            pltpu.VMEM((_BPW, D_MODEL), jnp.float32),
            pltpu.SemaphoreType.DMA,
        ],
    )(_sc_gather_body)
    return k(table, idx)


def kernel(inputs, mask, W_router, W1, b1, W2, b2):
    x = inputs.reshape(N_TOKENS, D_MODEL)
    logits_pad, sel2d, wtop, pos2d, meta = _router(x, W_router, mask)
    pos = pos2d.reshape(N_TOKENS)
    xs = _sc_scatter(x, pos)
    ys = _gmm(meta, xs, W1, b1, W2, b2)
    out = _sc_gather(ys, pos)
    results = (out * wtop).reshape(inputs.shape)
    return (results, logits_pad[:, :N_EXPERTS + 1], sel2d)
